# baseline (speedup 1.0000x reference)
import jax
import jax.numpy as jnp
from jax import lax
from jax.experimental import pallas as pl

_EPS = 1e-8
_THRESH = 0.9
_BB = 64


def _merge_block_kernel(slots_ref, final_ref, smask_ref):
    BB, S, D = slots_ref.shape
    N = BB * S
    X = slots_ref[...].reshape(N, D)
    lane = lax.broadcasted_iota(jnp.int32, (N, S), 1)
    rowid = lax.broadcasted_iota(jnp.int32, (N, 1), 0) & (S - 1)
    inv = lax.rsqrt(jnp.sum(X * X, axis=1, keepdims=True))
    Y = X * inv
    gs = []
    for b in range(BB):
        yb = Y[b * S:(b + 1) * S]
        gs.append(lax.dot_general(yb, yb, (((1,), (1,)), ((), ())),
                                  preferred_element_type=jnp.float32))
    G = jnp.concatenate(gs, axis=0)
    maskb = G > _THRESH
    maskf = maskb.astype(jnp.float32)
    count = jnp.sum(maskf, axis=1, keepdims=True)
    multi_f = (count > 1.0).astype(jnp.float32)
    fi = jnp.min(jnp.where(maskb, lane, S), axis=1, keepdims=True)
    cols = jnp.concatenate([multi_f, fi.astype(jnp.float32)], axis=1)
    mrows, frows = [], []
    for b in range(BB):
        t = jnp.transpose(cols[b * S:(b + 1) * S])
        mrows.append(jnp.broadcast_to(t[0:1, :], (S, S)))
        frows.append(jnp.broadcast_to(t[1:2, :], (S, S)))
    MR = jnp.concatenate(mrows, axis=0)
    FR = jnp.concatenate(frows, axis=0)
    rowid_f = rowid.astype(jnp.float32)
    zc = maskf * MR * (rowid_f != FR).astype(jnp.float32)
    smask_col = 1.0 - jnp.max(zc, axis=1, keepdims=True)
    wm = maskf * MR
    slw = jnp.max(jnp.where(wm > 0.5, lane, -1), axis=1, keepdims=True)
    s_last = jnp.where(multi_f > 0.5, slw, jnp.maximum(slw, rowid))
    oh = (s_last == lane).astype(jnp.float32)
    eye_st = (rowid == lane).astype(jnp.float32)
    mhat = multi_f * (maskf / (count + _EPS)) + (1.0 - multi_f) * eye_st
    for b in range(BB):
        sl = slice(b * S, (b + 1) * S)
        w = lax.dot_general(oh[sl], mhat[sl], (((1,), (0,)), ((), ())),
                            preferred_element_type=jnp.float32)
        final_ref[b] = lax.dot_general(w, X[sl], (((1,), (0,)), ((), ())),
                                       preferred_element_type=jnp.float32)
    smask_ref[...] = smask_col.reshape(BB, S)


def kernel(slots):
    B, S, D = slots.shape
    grid = (B // _BB,)
    final, smask = pl.pallas_call(
        _merge_block_kernel,
        grid=grid,
        in_specs=[pl.BlockSpec((_BB, S, D), lambda i: (i, 0, 0))],
        out_specs=[
            pl.BlockSpec((_BB, S, D), lambda i: (i, 0, 0)),
            pl.BlockSpec((_BB, S), lambda i: (i, 0)),
        ],
        out_shape=[
            jax.ShapeDtypeStruct((B, S, D), slots.dtype),
            jax.ShapeDtypeStruct((B, S), slots.dtype),
        ],
    )(slots)
    return final, smask


# speedup vs baseline: 1.3261x; 1.3261x over previous
import jax
import jax.numpy as jnp
from jax import lax
from jax.experimental import pallas as pl

_EPS = 1e-8
_THRESH = 0.9
_BB = 64


def _merge_block_kernel(slots_ref, final_ref, smask_ref):
    BB, S, D = slots_ref.shape
    N = BB * S
    X = slots_ref[...].reshape(N, D)
    lane = lax.broadcasted_iota(jnp.int32, (N, S), 1)
    lane_f = lane.astype(jnp.float32)
    rowid = lax.broadcasted_iota(jnp.int32, (N, 1), 0) & (S - 1)
    rowid_f = rowid.astype(jnp.float32)
    inv = lax.rsqrt(jnp.sum(X * X, axis=1, keepdims=True))
    Y = X * inv
    gs = []
    for b in range(BB):
        yb = Y[b * S:(b + 1) * S]
        gs.append(lax.dot_general(yb, yb, (((1,), (1,)), ((), ())),
                                  preferred_element_type=jnp.float32))
    G = jnp.concatenate(gs, axis=0)
    maskf = (G > _THRESH).astype(jnp.float32)
    count = jnp.sum(maskf, axis=1, keepdims=True)
    multi_f = (count > 1.0).astype(jnp.float32)
    mhat = maskf * (1.0 / (count + _EPS))
    fi = float(S) - jnp.max(maskf * (S - lane_f), axis=1, keepdims=True)
    cols = jnp.concatenate([multi_f, fi], axis=1)
    mrows, frows = [], []
    for b in range(BB):
        t = jnp.transpose(cols[b * S:(b + 1) * S])
        mrows.append(jnp.broadcast_to(t[0:1, :], (S, S)))
        frows.append(jnp.broadcast_to(t[1:2, :], (S, S)))
    MR = jnp.concatenate(mrows, axis=0)
    FR = jnp.concatenate(frows, axis=0)
    wm = maskf * MR
    zc = wm * (rowid_f != FR).astype(jnp.float32)
    smask_col = 1.0 - jnp.max(zc, axis=1, keepdims=True)
    slw = jnp.max(wm * (lane_f + 1.0), axis=1, keepdims=True) - 1.0
    s_last = jnp.maximum(slw, rowid_f)
    oh = (s_last == lane_f).astype(jnp.float32)
    for b in range(BB):
        sl = slice(b * S, (b + 1) * S)
        w = lax.dot_general(oh[sl], mhat[sl], (((1,), (0,)), ((), ())),
                            preferred_element_type=jnp.float32)
        final_ref[b] = lax.dot_general(w, X[sl], (((1,), (0,)), ((), ())),
                                       preferred_element_type=jnp.float32)
    smask_ref[...] = smask_col.reshape(BB, S)


def kernel(slots):
    B, S, D = slots.shape
    grid = (B // _BB,)
    final, smask = pl.pallas_call(
        _merge_block_kernel,
        grid=grid,
        in_specs=[pl.BlockSpec((_BB, S, D), lambda i: (i, 0, 0))],
        out_specs=[
            pl.BlockSpec((_BB, S, D), lambda i: (i, 0, 0)),
            pl.BlockSpec((_BB, S), lambda i: (i, 0)),
        ],
        out_shape=[
            jax.ShapeDtypeStruct((B, S, D), slots.dtype),
            jax.ShapeDtypeStruct((B, S), slots.dtype),
        ],
    )(slots)
    return final, smask


# speedup vs baseline: 1.3956x; 1.0524x over previous
import jax
import jax.numpy as jnp
from jax import lax
from jax.experimental import pallas as pl

_EPS = 1e-8
_THRESH = 0.9
_BB = 64


def _merge_block_kernel(slots_ref, final_ref, smask_ref):
    BB, S, D = slots_ref.shape
    N = BB * S
    X = slots_ref[...].reshape(N, D)
    lane = lax.broadcasted_iota(jnp.int32, (N, S), 1)
    lane_f = lane.astype(jnp.float32)
    rowid = lax.broadcasted_iota(jnp.int32, (N, 1), 0) & (S - 1)
    rowid_f = rowid.astype(jnp.float32)
    inv = lax.rsqrt(jnp.sum(X * X, axis=1, keepdims=True))
    Y = X * inv
    gs = []
    for b in range(BB):
        yb = Y[b * S:(b + 1) * S]
        gs.append(lax.dot_general(yb, yb, (((1,), (1,)), ((), ())),
                                  preferred_element_type=jnp.float32))
    G = jnp.concatenate(gs, axis=0)
    maskf = (G > _THRESH).astype(jnp.float32)
    count = jnp.sum(maskf, axis=1, keepdims=True)
    multi_f = (count > 1.0).astype(jnp.float32)
    mhat = maskf * (1.0 / (count + _EPS))
    fi = float(S) - jnp.max(maskf * (S - lane_f), axis=1, keepdims=True)
    F = (fi == lane_f).astype(jnp.float32)
    ones_col = jnp.ones((S, 1), jnp.float32)
    mrs, zcs = [], []
    for b in range(BB):
        sl = slice(b * S, (b + 1) * S)
        mb = multi_f[sl]
        mrs.append(lax.dot_general(ones_col, mb, (((1,), (1,)), ((), ())),
                                   preferred_element_type=jnp.float32))
        zcs.append(lax.dot_general(F[sl], mb, (((0,), (0,)), ((), ())),
                                   preferred_element_type=jnp.float32))
    MR = jnp.concatenate(mrs, axis=0)
    ZC = jnp.concatenate(zcs, axis=0)
    wm = maskf * MR
    covered = jnp.sum(wm, axis=1, keepdims=True)
    smask_col = (covered == ZC).astype(jnp.float32)
    slw = jnp.max(wm * (lane_f + 1.0), axis=1, keepdims=True) - 1.0
    s_last = jnp.maximum(slw, rowid_f)
    oh = (s_last == lane_f).astype(jnp.float32)
    for b in range(BB):
        sl = slice(b * S, (b + 1) * S)
        w = lax.dot_general(oh[sl], mhat[sl], (((1,), (0,)), ((), ())),
                            preferred_element_type=jnp.float32)
        final_ref[b] = lax.dot_general(w, X[sl], (((1,), (0,)), ((), ())),
                                       preferred_element_type=jnp.float32)
    smask_ref[...] = smask_col.reshape(BB, S)


def kernel(slots):
    B, S, D = slots.shape
    grid = (B // _BB,)
    final, smask = pl.pallas_call(
        _merge_block_kernel,
        grid=grid,
        in_specs=[pl.BlockSpec((_BB, S, D), lambda i: (i, 0, 0))],
        out_specs=[
            pl.BlockSpec((_BB, S, D), lambda i: (i, 0, 0)),
            pl.BlockSpec((_BB, S), lambda i: (i, 0)),
        ],
        out_shape=[
            jax.ShapeDtypeStruct((B, S, D), slots.dtype),
            jax.ShapeDtypeStruct((B, S), slots.dtype),
        ],
    )(slots)
    return final, smask


# speedup vs baseline: 1.6258x; 1.1650x over previous
import jax
import jax.numpy as jnp
from jax import lax
from jax.experimental import pallas as pl

_EPS = 1e-8
_THRESH = 0.9
_BB = 64


def _merge_block_kernel(slots_ref, final_ref, smask_ref):
    BB, S, D = slots_ref.shape
    N = BB * S
    S2 = 2 * S
    NP = N // S2
    X = slots_ref[...].reshape(N, D)
    lane = lax.broadcasted_iota(jnp.int32, (N, S2), 1)
    lane_f = lane.astype(jnp.float32)
    rowid = lax.broadcasted_iota(jnp.int32, (N, 1), 0) & (S2 - 1)
    rowid_f = rowid.astype(jnp.float32)
    valid_f = ((rowid & S) == (lane & S)).astype(jnp.float32)
    inv = lax.rsqrt(jnp.sum(X * X, axis=1, keepdims=True))
    Y = X * inv
    gs = []
    for p in range(NP):
        yp = Y[p * S2:(p + 1) * S2]
        gs.append(lax.dot_general(yp, yp, (((1,), (1,)), ((), ())),
                                  preferred_element_type=jnp.float32))
    G = jnp.concatenate(gs, axis=0)
    maskf = (G > _THRESH).astype(jnp.float32) * valid_f
    count = jnp.sum(maskf, axis=1, keepdims=True)
    multi_f = (count > 1.0).astype(jnp.float32)
    mhat = maskf * (1.0 / (count + _EPS))
    fi = float(S2) - jnp.max(maskf * (S2 - lane_f), axis=1, keepdims=True)
    F = (fi == lane_f).astype(jnp.float32)
    ones_col = jnp.ones((S2, 1), jnp.float32)
    mrs, zcs = [], []
    for p in range(NP):
        sl = slice(p * S2, (p + 1) * S2)
        mb = multi_f[sl]
        mrs.append(lax.dot_general(ones_col, mb, (((1,), (1,)), ((), ())),
                                   preferred_element_type=jnp.float32))
        zcs.append(lax.dot_general(F[sl], mb, (((0,), (0,)), ((), ())),
                                   preferred_element_type=jnp.float32))
    MR = jnp.concatenate(mrs, axis=0)
    ZC = jnp.concatenate(zcs, axis=0)
    wm = maskf * MR
    covered = jnp.sum(wm, axis=1, keepdims=True)
    smask_col = (covered == ZC).astype(jnp.float32)
    slw = jnp.max(wm * (lane_f + 1.0), axis=1, keepdims=True) - 1.0
    s_last = jnp.maximum(slw, rowid_f)
    oh = (s_last == lane_f).astype(jnp.float32)
    for p in range(NP):
        sl = slice(p * S2, (p + 1) * S2)
        w = lax.dot_general(oh[sl], mhat[sl], (((1,), (0,)), ((), ())),
                            preferred_element_type=jnp.float32)
        out = lax.dot_general(w, X[sl], (((1,), (0,)), ((), ())),
                              preferred_element_type=jnp.float32)
        final_ref[2 * p] = out[:S]
        final_ref[2 * p + 1] = out[S:]
    smask_ref[...] = smask_col.reshape(BB, S)


def kernel(slots):
    B, S, D = slots.shape
    grid = (B // _BB,)
    final, smask = pl.pallas_call(
        _merge_block_kernel,
        grid=grid,
        in_specs=[pl.BlockSpec((_BB, S, D), lambda i: (i, 0, 0))],
        out_specs=[
            pl.BlockSpec((_BB, S, D), lambda i: (i, 0, 0)),
            pl.BlockSpec((_BB, S), lambda i: (i, 0)),
        ],
        out_shape=[
            jax.ShapeDtypeStruct((B, S, D), slots.dtype),
            jax.ShapeDtypeStruct((B, S), slots.dtype),
        ],
    )(slots)
    return final, smask
